# initial kernel scaffold (unmeasured)
import jax
import jax.numpy as jnp
from jax import lax
from jax.experimental import pallas as pl
from jax.experimental.pallas import tpu as pltpu


def kernel(
    x,
):
    def body(*refs):
        pass

    out_shape = jax.ShapeDtypeStruct(..., jnp.float32)
    return pl.pallas_call(body, out_shape=out_shape)(...)


# baseline (device time: 54944 ns/iter reference)
import jax
import jax.numpy as jnp
from jax import lax
from jax.experimental import pallas as pl
from jax.experimental.pallas import tpu as pltpu


def kernel(x):
    _, m, n2 = x.shape
    n = n2 // 2

    def body(x_ref, out_ref, send_buf, recv_buf, send_sem, recv_sem):
        my_x = lax.axis_index("x")
        my_y = lax.axis_index("y")
        my_z = lax.axis_index("z")
        peer = (my_x, my_y, 1 - my_z)

        barrier_sem = pltpu.get_barrier_semaphore()
        pl.semaphore_signal(
            barrier_sem, inc=1, device_id=peer,
            device_id_type=pl.DeviceIdType.MESH,
        )
        pl.semaphore_wait(barrier_sem, 1)

        @pl.when(my_z == 0)
        def _():
            send_buf[:, :] = x_ref[0, :, n:]

        @pl.when(my_z == 1)
        def _():
            send_buf[:, :] = x_ref[0, :, :n]

        rdma = pltpu.make_async_remote_copy(
            src_ref=send_buf,
            dst_ref=recv_buf,
            send_sem=send_sem,
            recv_sem=recv_sem,
            device_id=peer,
            device_id_type=pl.DeviceIdType.MESH,
        )
        rdma.start()
        rdma.wait()

        @pl.when(my_z == 0)
        def _():
            out_ref[:, :] = x_ref[0, :, :n] + recv_buf[:, :]

        @pl.when(my_z == 1)
        def _():
            out_ref[:, :] = x_ref[0, :, n:] + recv_buf[:, :]

    return pl.pallas_call(
        body,
        out_shape=jax.ShapeDtypeStruct((m, n), jnp.float32),
        in_specs=[pl.BlockSpec(memory_space=pltpu.VMEM)],
        out_specs=pl.BlockSpec(memory_space=pltpu.VMEM),
        scratch_shapes=[
            pltpu.VMEM((m, n), jnp.float32),
            pltpu.VMEM((m, n), jnp.float32),
            pltpu.SemaphoreType.DMA,
            pltpu.SemaphoreType.DMA,
        ],
        compiler_params=pltpu.CompilerParams(collective_id=0),
    )(x)


# device time: 38519 ns/iter; 1.4264x vs baseline; 1.4264x over previous
import jax
import jax.numpy as jnp
from jax import lax
from jax.experimental import pallas as pl
from jax.experimental.pallas import tpu as pltpu

Q = 4
C = 4


def kernel(x):
    _, m, n2 = x.shape
    n = n2 // 2
    qm = m // Q
    ck = qm // C

    def body(x_ref, out_ref, g_ref,
             zs_sems, zr_sems, xs_sems, xr_sems, ys_sems, yr_sems):
        my_x = lax.axis_index("x")
        my_y = lax.axis_index("y")
        my_z = lax.axis_index("z")

        q_row = (2 * my_x + my_y) * qm
        qx_row = (2 * (1 - my_x) + my_y) * qm
        qy_row = (2 * my_x + (1 - my_y)) * qm
        qxy_row = (2 * (1 - my_x) + (1 - my_y)) * qm

        my_col = my_z * n
        peer_col = (1 - my_z) * n

        z_peer = (my_x, my_y, 1 - my_z)
        x_nbr = (1 - my_x, my_y, my_z)
        y_nbr = (my_x, 1 - my_y, my_z)

        def rcopy(src, dst, ssem, rsem, dev):
            return pltpu.make_async_remote_copy(
                src_ref=src, dst_ref=dst, send_sem=ssem, recv_sem=rsem,
                device_id=dev, device_id_type=pl.DeviceIdType.MESH,
            )

        barrier_sem = pltpu.get_barrier_semaphore()
        for nbr in (z_peer, x_nbr, y_nbr):
            pl.semaphore_signal(
                barrier_sem, inc=1, device_id=nbr,
                device_id_type=pl.DeviceIdType.MESH,
            )
        pl.semaphore_wait(barrier_sem, 3)

        z_send, x_send, y_send0, y_send1 = [], [], [], []
        for c in range(C):
            r = q_row + c * ck
            rd = rcopy(
                x_ref.at[0, pl.ds(r, ck), pl.ds(peer_col, n)],
                g_ref.at[pl.ds(r, ck), :],
                zs_sems.at[c], zr_sems.at[c], z_peer,
            )
            rd.start()
            z_send.append(rd)

        z_recv = [
            rcopy(
                x_ref.at[0, pl.ds(q_row + c * ck, ck), pl.ds(peer_col, n)],
                g_ref.at[pl.ds(q_row + c * ck, ck), :],
                zs_sems.at[c], zr_sems.at[c], z_peer,
            )
            for c in range(C)
        ]
        x_recv = [
            rcopy(
                x_ref.at[0, pl.ds(qx_row + c * ck, ck), pl.ds(peer_col, n)],
                g_ref.at[pl.ds(qx_row + c * ck, ck), :],
                xs_sems.at[c], xr_sems.at[c], x_nbr,
            )
            for c in range(C)
        ]
        y_recv0 = [
            rcopy(
                x_ref.at[0, pl.ds(qy_row + c * ck, ck), pl.ds(peer_col, n)],
                g_ref.at[pl.ds(qy_row + c * ck, ck), :],
                ys_sems.at[0, c], yr_sems.at[0, c], y_nbr,
            )
            for c in range(C)
        ]
        y_recv1 = [
            rcopy(
                x_ref.at[0, pl.ds(qxy_row + c * ck, ck), pl.ds(peer_col, n)],
                g_ref.at[pl.ds(qxy_row + c * ck, ck), :],
                ys_sems.at[1, c], yr_sems.at[1, c], y_nbr,
            )
            for c in range(C)
        ]

        for c in range(C):
            r = q_row + c * ck
            z_recv[c].wait_recv()
            rd = rcopy(
                g_ref.at[pl.ds(r, ck), :], g_ref.at[pl.ds(r, ck), :],
                xs_sems.at[c], xr_sems.at[c], x_nbr,
            )
            rd.start()
            x_send.append(rd)
            rd = rcopy(
                g_ref.at[pl.ds(r, ck), :], g_ref.at[pl.ds(r, ck), :],
                ys_sems.at[0, c], yr_sems.at[0, c], y_nbr,
            )
            rd.start()
            y_send0.append(rd)

        for c in range(C):
            r = qx_row + c * ck
            x_recv[c].wait_recv()
            rd = rcopy(
                g_ref.at[pl.ds(r, ck), :], g_ref.at[pl.ds(r, ck), :],
                ys_sems.at[1, c], yr_sems.at[1, c], y_nbr,
            )
            rd.start()
            y_send1.append(rd)

        for r in (q_row, qx_row):
            out_ref[pl.ds(r, qm), :] = (
                x_ref[0, pl.ds(r, qm), pl.ds(my_col, n)]
                + g_ref[pl.ds(r, qm), :]
            )

        for c in range(C):
            y_recv0[c].wait_recv()
            y_recv1[c].wait_recv()
        for r in (qy_row, qxy_row):
            out_ref[pl.ds(r, qm), :] = (
                x_ref[0, pl.ds(r, qm), pl.ds(my_col, n)]
                + g_ref[pl.ds(r, qm), :]
            )

        for rd in z_send + x_send + y_send0 + y_send1:
            rd.wait_send()

    return pl.pallas_call(
        body,
        out_shape=jax.ShapeDtypeStruct((m, n), jnp.float32),
        in_specs=[pl.BlockSpec(memory_space=pltpu.VMEM)],
        out_specs=pl.BlockSpec(memory_space=pltpu.VMEM),
        scratch_shapes=[
            pltpu.VMEM((m, n), jnp.float32),
            pltpu.SemaphoreType.DMA((C,)),
            pltpu.SemaphoreType.DMA((C,)),
            pltpu.SemaphoreType.DMA((C,)),
            pltpu.SemaphoreType.DMA((C,)),
            pltpu.SemaphoreType.DMA((2, C)),
            pltpu.SemaphoreType.DMA((2, C)),
        ],
        compiler_params=pltpu.CompilerParams(collective_id=0),
    )(x)


# device time: 37484 ns/iter; 1.4658x vs baseline; 1.0276x over previous
import jax
import jax.numpy as jnp
from jax import lax
from jax.experimental import pallas as pl
from jax.experimental.pallas import tpu as pltpu

Q = 4
C = 8


def kernel(x):
    _, m, n2 = x.shape
    n = n2 // 2
    qm = m // Q
    ck = qm // C

    def body(x_ref, out_ref, g_ref, sb_ref,
             zs_sems, zr_sems, xs_sems, xr_sems, ys_sems, yr_sems):
        my_x = lax.axis_index("x")
        my_y = lax.axis_index("y")
        my_z = lax.axis_index("z")

        q_row = (2 * my_x + my_y) * qm
        qx_row = (2 * (1 - my_x) + my_y) * qm
        qy_row = (2 * my_x + (1 - my_y)) * qm
        qxy_row = (2 * (1 - my_x) + (1 - my_y)) * qm

        my_col = my_z * n
        peer_col = (1 - my_z) * n

        z_peer = (my_x, my_y, 1 - my_z)
        x_nbr = (1 - my_x, my_y, my_z)
        y_nbr = (my_x, 1 - my_y, my_z)

        def rcopy(src, dst, ssem, rsem, dev):
            return pltpu.make_async_remote_copy(
                src_ref=src, dst_ref=dst, send_sem=ssem, recv_sem=rsem,
                device_id=dev, device_id_type=pl.DeviceIdType.MESH,
            )

        barrier_sem = pltpu.get_barrier_semaphore()
        for nbr in (z_peer, x_nbr, y_nbr):
            pl.semaphore_signal(
                barrier_sem, inc=1, device_id=nbr,
                device_id_type=pl.DeviceIdType.MESH,
            )
        pl.semaphore_wait(barrier_sem, 3)

        sb_ref[:, :] = x_ref[0, pl.ds(q_row, qm), pl.ds(peer_col, n)]

        z_send, x_send, y_send0, y_send1 = [], [], [], []
        for c in range(C):
            r = q_row + c * ck
            rd = rcopy(
                sb_ref.at[pl.ds(c * ck, ck), :],
                g_ref.at[pl.ds(r, ck), :],
                zs_sems.at[c], zr_sems.at[c], z_peer,
            )
            rd.start()
            z_send.append(rd)

        z_recv = [
            rcopy(
                x_ref.at[0, pl.ds(q_row + c * ck, ck), pl.ds(peer_col, n)],
                g_ref.at[pl.ds(q_row + c * ck, ck), :],
                zs_sems.at[c], zr_sems.at[c], z_peer,
            )
            for c in range(C)
        ]
        x_recv = [
            rcopy(
                x_ref.at[0, pl.ds(qx_row + c * ck, ck), pl.ds(peer_col, n)],
                g_ref.at[pl.ds(qx_row + c * ck, ck), :],
                xs_sems.at[c], xr_sems.at[c], x_nbr,
            )
            for c in range(C)
        ]
        y_recv0 = [
            rcopy(
                x_ref.at[0, pl.ds(qy_row + c * ck, ck), pl.ds(peer_col, n)],
                g_ref.at[pl.ds(qy_row + c * ck, ck), :],
                ys_sems.at[0, c], yr_sems.at[0, c], y_nbr,
            )
            for c in range(C)
        ]
        y_recv1 = [
            rcopy(
                x_ref.at[0, pl.ds(qxy_row + c * ck, ck), pl.ds(peer_col, n)],
                g_ref.at[pl.ds(qxy_row + c * ck, ck), :],
                ys_sems.at[1, c], yr_sems.at[1, c], y_nbr,
            )
            for c in range(C)
        ]

        for c in range(C):
            r = q_row + c * ck
            z_recv[c].wait_recv()
            rd = rcopy(
                g_ref.at[pl.ds(r, ck), :], g_ref.at[pl.ds(r, ck), :],
                xs_sems.at[c], xr_sems.at[c], x_nbr,
            )
            rd.start()
            x_send.append(rd)
            rd = rcopy(
                g_ref.at[pl.ds(r, ck), :], g_ref.at[pl.ds(r, ck), :],
                ys_sems.at[0, c], yr_sems.at[0, c], y_nbr,
            )
            rd.start()
            y_send0.append(rd)
            if c >= 1:
                rx = qx_row + (c - 1) * ck
                x_recv[c - 1].wait_recv()
                rd = rcopy(
                    g_ref.at[pl.ds(rx, ck), :], g_ref.at[pl.ds(rx, ck), :],
                    ys_sems.at[1, c - 1], yr_sems.at[1, c - 1], y_nbr,
                )
                rd.start()
                y_send1.append(rd)
        rx = qx_row + (C - 1) * ck
        x_recv[C - 1].wait_recv()
        rd = rcopy(
            g_ref.at[pl.ds(rx, ck), :], g_ref.at[pl.ds(rx, ck), :],
            ys_sems.at[1, C - 1], yr_sems.at[1, C - 1], y_nbr,
        )
        rd.start()
        y_send1.append(rd)

        for r in (q_row, qx_row):
            out_ref[pl.ds(r, qm), :] = (
                x_ref[0, pl.ds(r, qm), pl.ds(my_col, n)]
                + g_ref[pl.ds(r, qm), :]
            )

        for c in range(C):
            y_recv0[c].wait_recv()
            y_recv1[c].wait_recv()
        for r in (qy_row, qxy_row):
            out_ref[pl.ds(r, qm), :] = (
                x_ref[0, pl.ds(r, qm), pl.ds(my_col, n)]
                + g_ref[pl.ds(r, qm), :]
            )

        for rd in z_send + x_send + y_send0 + y_send1:
            rd.wait_send()

    return pl.pallas_call(
        body,
        out_shape=jax.ShapeDtypeStruct((m, n), jnp.float32),
        in_specs=[pl.BlockSpec(memory_space=pltpu.VMEM)],
        out_specs=pl.BlockSpec(memory_space=pltpu.VMEM),
        scratch_shapes=[
            pltpu.VMEM((m, n), jnp.float32),
            pltpu.VMEM((m // Q, n), jnp.float32),
            pltpu.SemaphoreType.DMA((C,)),
            pltpu.SemaphoreType.DMA((C,)),
            pltpu.SemaphoreType.DMA((C,)),
            pltpu.SemaphoreType.DMA((C,)),
            pltpu.SemaphoreType.DMA((2, C)),
            pltpu.SemaphoreType.DMA((2, C)),
        ],
        compiler_params=pltpu.CompilerParams(collective_id=0),
    )(x)


# device time: 37219 ns/iter; 1.4762x vs baseline; 1.0071x over previous
import jax
import jax.numpy as jnp
from jax import lax
from jax.experimental import pallas as pl
from jax.experimental.pallas import tpu as pltpu

Q = 4
C = 8


def kernel(x):
    _, m, n2 = x.shape
    n = n2 // 2
    qm = m // Q
    ck = qm // C

    def body(x_ref, out_ref, g_ref, sb_ref,
             zs_sems, zr_sems, xs_sems, xr_sems, ys_sems, yr_sems):
        my_x = lax.axis_index("x")
        my_y = lax.axis_index("y")
        my_z = lax.axis_index("z")

        def rows(base, c=0):
            return pl.ds(pl.multiple_of(base + c * ck, ck), ck)

        q_row = (2 * my_x + my_y) * qm
        qx_row = (2 * (1 - my_x) + my_y) * qm
        qy_row = (2 * my_x + (1 - my_y)) * qm
        qxy_row = (2 * (1 - my_x) + (1 - my_y)) * qm

        my_col = pl.multiple_of(my_z * n, n)
        peer_col = pl.multiple_of((1 - my_z) * n, n)

        z_peer = (my_x, my_y, 1 - my_z)
        x_nbr = (1 - my_x, my_y, my_z)
        y_nbr = (my_x, 1 - my_y, my_z)

        def rcopy(src, dst, ssem, rsem, dev):
            return pltpu.make_async_remote_copy(
                src_ref=src, dst_ref=dst, send_sem=ssem, recv_sem=rsem,
                device_id=dev, device_id_type=pl.DeviceIdType.MESH,
            )

        def add_chunk(base, c):
            out_ref[rows(base, c), :] = (
                x_ref[0, rows(base, c), pl.ds(my_col, n)]
                + g_ref[rows(base, c), :]
            )

        barrier_sem = pltpu.get_barrier_semaphore()
        for nbr in (z_peer, x_nbr, y_nbr):
            pl.semaphore_signal(
                barrier_sem, inc=1, device_id=nbr,
                device_id_type=pl.DeviceIdType.MESH,
            )
        pl.semaphore_wait(barrier_sem, 3)

        sb_ref[:, :] = x_ref[0, pl.ds(q_row, qm), pl.ds(peer_col, n)]

        z_send, x_send, y_send0, y_send1 = [], [], [], []
        for c in range(C):
            rd = rcopy(
                sb_ref.at[rows(0, c), :], g_ref.at[rows(q_row, c), :],
                zs_sems.at[c], zr_sems.at[c], z_peer,
            )
            rd.start()
            z_send.append(rd)

        z_recv = [
            rcopy(sb_ref.at[rows(0, c), :], g_ref.at[rows(q_row, c), :],
                  zs_sems.at[c], zr_sems.at[c], z_peer)
            for c in range(C)
        ]
        x_recv = [
            rcopy(sb_ref.at[rows(0, c), :], g_ref.at[rows(qx_row, c), :],
                  xs_sems.at[c], xr_sems.at[c], x_nbr)
            for c in range(C)
        ]
        y_recv0 = [
            rcopy(sb_ref.at[rows(0, c), :], g_ref.at[rows(qy_row, c), :],
                  ys_sems.at[0, c], yr_sems.at[0, c], y_nbr)
            for c in range(C)
        ]
        y_recv1 = [
            rcopy(sb_ref.at[rows(0, c), :], g_ref.at[rows(qxy_row, c), :],
                  ys_sems.at[1, c], yr_sems.at[1, c], y_nbr)
            for c in range(C)
        ]

        for c in range(C):
            z_recv[c].wait_recv()
            rd = rcopy(
                g_ref.at[rows(q_row, c), :], g_ref.at[rows(q_row, c), :],
                xs_sems.at[c], xr_sems.at[c], x_nbr,
            )
            rd.start()
            x_send.append(rd)
            rd = rcopy(
                g_ref.at[rows(q_row, c), :], g_ref.at[rows(q_row, c), :],
                ys_sems.at[0, c], yr_sems.at[0, c], y_nbr,
            )
            rd.start()
            y_send0.append(rd)
            add_chunk(q_row, c)
            if c >= 1:
                x_recv[c - 1].wait_recv()
                rd = rcopy(
                    g_ref.at[rows(qx_row, c - 1), :],
                    g_ref.at[rows(qx_row, c - 1), :],
                    ys_sems.at[1, c - 1], yr_sems.at[1, c - 1], y_nbr,
                )
                rd.start()
                y_send1.append(rd)
                add_chunk(qx_row, c - 1)
        x_recv[C - 1].wait_recv()
        rd = rcopy(
            g_ref.at[rows(qx_row, C - 1), :], g_ref.at[rows(qx_row, C - 1), :],
            ys_sems.at[1, C - 1], yr_sems.at[1, C - 1], y_nbr,
        )
        rd.start()
        y_send1.append(rd)
        add_chunk(qx_row, C - 1)

        for c in range(C):
            y_recv0[c].wait_recv()
            add_chunk(qy_row, c)
            y_recv1[c].wait_recv()
            add_chunk(qxy_row, c)

        for rd in z_send + x_send + y_send0 + y_send1:
            rd.wait_send()

    return pl.pallas_call(
        body,
        out_shape=jax.ShapeDtypeStruct((m, n), jnp.float32),
        in_specs=[pl.BlockSpec(memory_space=pltpu.VMEM)],
        out_specs=pl.BlockSpec(memory_space=pltpu.VMEM),
        scratch_shapes=[
            pltpu.VMEM((m, n), jnp.float32),
            pltpu.VMEM((m // Q, n), jnp.float32),
            pltpu.SemaphoreType.DMA((C,)),
            pltpu.SemaphoreType.DMA((C,)),
            pltpu.SemaphoreType.DMA((C,)),
            pltpu.SemaphoreType.DMA((C,)),
            pltpu.SemaphoreType.DMA((2, C)),
            pltpu.SemaphoreType.DMA((2, C)),
        ],
        compiler_params=pltpu.CompilerParams(collective_id=0),
    )(x)


# device time: 33061 ns/iter; 1.6619x vs baseline; 1.1258x over previous
import jax
import jax.numpy as jnp
from jax import lax
from jax.experimental import pallas as pl
from jax.experimental.pallas import tpu as pltpu

Q = 4
C = 8


def kernel(x):
    _, m, n2 = x.shape
    n = n2 // 2
    h = n // 2
    qm = m // Q
    ck = qm // C

    def body(x_ref, out_ref, g_ref, sb_ref,
             z_s, z_r, xq_s, xq_r, yq_s, yq_r, xr_s, xr_r, yr_s, yr_r):
        my_x = lax.axis_index("x")
        my_y = lax.axis_index("y")
        my_z = lax.axis_index("z")

        def rows(base, c=0):
            return pl.ds(pl.multiple_of(base + c * ck, ck), ck)

        colA = pl.ds(0, h)
        colB = pl.ds(h, h)

        q_row = (2 * my_x + my_y) * qm
        qx_row = (2 * (1 - my_x) + my_y) * qm
        qy_row = (2 * my_x + (1 - my_y)) * qm
        qxy_row = (2 * (1 - my_x) + (1 - my_y)) * qm

        my_col = pl.multiple_of(my_z * n, n)
        peer_col = pl.multiple_of((1 - my_z) * n, n)

        z_peer = (my_x, my_y, 1 - my_z)
        x_nbr = (1 - my_x, my_y, my_z)
        y_nbr = (my_x, 1 - my_y, my_z)

        def rcopy(src, dst, ssem, rsem, dev):
            return pltpu.make_async_remote_copy(
                src_ref=src, dst_ref=dst, send_sem=ssem, recv_sem=rsem,
                device_id=dev, device_id_type=pl.DeviceIdType.MESH,
            )

        def add_chunk(base, c):
            out_ref[rows(base, c), :] = (
                x_ref[0, rows(base, c), pl.ds(my_col, n)]
                + g_ref[rows(base, c), :]
            )

        barrier_sem = pltpu.get_barrier_semaphore()
        for nbr in (z_peer, x_nbr, y_nbr):
            pl.semaphore_signal(
                barrier_sem, inc=1, device_id=nbr,
                device_id_type=pl.DeviceIdType.MESH,
            )
        pl.semaphore_wait(barrier_sem, 3)

        sb_ref[:, :] = x_ref[0, pl.ds(q_row, qm), pl.ds(peer_col, n)]

        sends = []
        for c in range(C):
            rd = rcopy(
                sb_ref.at[rows(0, c), :], g_ref.at[rows(q_row, c), :],
                z_s.at[c], z_r.at[c], z_peer,
            )
            rd.start()
            sends.append(rd)

        z_recv = [
            rcopy(sb_ref.at[rows(0, c), :], g_ref.at[rows(q_row, c), :],
                  z_s.at[c], z_r.at[c], z_peer)
            for c in range(C)
        ]
        xq_recv = [
            rcopy(sb_ref.at[rows(0, c), :], g_ref.at[rows(qx_row, c), :],
                  xq_s.at[c], xq_r.at[c], x_nbr)
            for c in range(C)
        ]
        yq_recv = [
            rcopy(sb_ref.at[rows(0, c), :], g_ref.at[rows(qy_row, c), :],
                  yq_s.at[c], yq_r.at[c], y_nbr)
            for c in range(C)
        ]
        yr_recv = [
            rcopy(sb_ref.at[rows(0, c), colA], g_ref.at[rows(qxy_row, c), colA],
                  yr_s.at[c], yr_r.at[c], y_nbr)
            for c in range(C)
        ]
        xr_recv = [
            rcopy(sb_ref.at[rows(0, c), colB], g_ref.at[rows(qxy_row, c), colB],
                  xr_s.at[c], xr_r.at[c], x_nbr)
            for c in range(C)
        ]

        def fwd_own(c):
            for ssem, rsem, dev in ((xq_s, xq_r, x_nbr), (yq_s, yq_r, y_nbr)):
                rd = rcopy(
                    g_ref.at[rows(q_row, c), :], g_ref.at[rows(q_row, c), :],
                    ssem.at[c], rsem.at[c], dev,
                )
                rd.start()
                sends.append(rd)

        def relay_qx(c):
            rd = rcopy(
                g_ref.at[rows(qx_row, c), colA], g_ref.at[rows(qx_row, c), colA],
                yr_s.at[c], yr_r.at[c], y_nbr,
            )
            rd.start()
            sends.append(rd)

        def relay_qy(c):
            rd = rcopy(
                g_ref.at[rows(qy_row, c), colB], g_ref.at[rows(qy_row, c), colB],
                xr_s.at[c], xr_r.at[c], x_nbr,
            )
            rd.start()
            sends.append(rd)

        for c in range(C):
            z_recv[c].wait_recv()
            fwd_own(c)
            add_chunk(q_row, c)
            if c >= 1:
                xq_recv[c - 1].wait_recv()
                relay_qx(c - 1)
                add_chunk(qx_row, c - 1)
                yq_recv[c - 1].wait_recv()
                relay_qy(c - 1)
                add_chunk(qy_row, c - 1)
        xq_recv[C - 1].wait_recv()
        relay_qx(C - 1)
        add_chunk(qx_row, C - 1)
        yq_recv[C - 1].wait_recv()
        relay_qy(C - 1)
        add_chunk(qy_row, C - 1)

        for c in range(C):
            yr_recv[c].wait_recv()
            xr_recv[c].wait_recv()
            add_chunk(qxy_row, c)

        for rd in sends:
            rd.wait_send()

    return pl.pallas_call(
        body,
        out_shape=jax.ShapeDtypeStruct((m, n), jnp.float32),
        in_specs=[pl.BlockSpec(memory_space=pltpu.VMEM)],
        out_specs=pl.BlockSpec(memory_space=pltpu.VMEM),
        scratch_shapes=[
            pltpu.VMEM((m, n), jnp.float32),
            pltpu.VMEM((m // Q, n), jnp.float32),
            pltpu.SemaphoreType.DMA((C,)),
            pltpu.SemaphoreType.DMA((C,)),
            pltpu.SemaphoreType.DMA((C,)),
            pltpu.SemaphoreType.DMA((C,)),
            pltpu.SemaphoreType.DMA((C,)),
            pltpu.SemaphoreType.DMA((C,)),
            pltpu.SemaphoreType.DMA((C,)),
            pltpu.SemaphoreType.DMA((C,)),
            pltpu.SemaphoreType.DMA((C,)),
            pltpu.SemaphoreType.DMA((C,)),
        ],
        compiler_params=pltpu.CompilerParams(collective_id=0),
    )(x)


# device time: 31352 ns/iter; 1.7525x vs baseline; 1.0545x over previous
import jax
import jax.numpy as jnp
from jax import lax
from jax.experimental import pallas as pl
from jax.experimental.pallas import tpu as pltpu

Q = 4
C = 8


def kernel(x):
    _, m, n2 = x.shape
    n = n2 // 2
    h = n // 2
    qm = m // Q
    ck = qm // C

    def body(x_ref, out_ref, g_ref, sb_ref,
             z_s, z_r, xq_s, xq_r, yq_s, yq_r, xr_s, xr_r, yr_s, yr_r):
        my_x = lax.axis_index("x")
        my_y = lax.axis_index("y")
        my_z = lax.axis_index("z")

        def rows(base, c=0):
            return pl.ds(pl.multiple_of(base + c * ck, ck), ck)

        colA = pl.ds(0, h)
        colB = pl.ds(h, h)

        q_row = (2 * my_x + my_y) * qm
        qx_row = (2 * (1 - my_x) + my_y) * qm
        qy_row = (2 * my_x + (1 - my_y)) * qm
        qxy_row = (2 * (1 - my_x) + (1 - my_y)) * qm

        my_col = pl.multiple_of(my_z * n, n)
        peer_col = pl.multiple_of((1 - my_z) * n, n)

        z_peer = (my_x, my_y, 1 - my_z)
        x_nbr = (1 - my_x, my_y, my_z)
        y_nbr = (my_x, 1 - my_y, my_z)

        def rcopy(src, dst, ssem, rsem, dev):
            return pltpu.make_async_remote_copy(
                src_ref=src, dst_ref=dst, send_sem=ssem, recv_sem=rsem,
                device_id=dev, device_id_type=pl.DeviceIdType.MESH,
            )

        def add_chunk(base, c):
            out_ref[rows(base, c), :] = (
                x_ref[0, rows(base, c), pl.ds(my_col, n)]
                + g_ref[rows(base, c), :]
            )

        barrier_sem = pltpu.get_barrier_semaphore()
        for nbr in (z_peer, x_nbr, y_nbr):
            pl.semaphore_signal(
                barrier_sem, inc=1, device_id=nbr,
                device_id_type=pl.DeviceIdType.MESH,
            )
        pl.semaphore_wait(barrier_sem, 3)

        sb_ref[:, :] = x_ref[0, pl.ds(q_row, qm), pl.ds(peer_col, n)]

        sends = []
        for c in range(C):
            rd = rcopy(
                sb_ref.at[rows(0, c), :], g_ref.at[rows(q_row, c), :],
                z_s.at[c], z_r.at[c], z_peer,
            )
            rd.start()
            sends.append(rd)
        for i, t_row in enumerate((qx_row, qy_row, qxy_row)):
            rd = rcopy(
                x_ref.at[0, rows(t_row, 0), pl.ds(peer_col, n)],
                g_ref.at[rows(t_row, 0), :],
                z_s.at[C + i], z_r.at[C + i], z_peer,
            )
            rd.start()
            sends.append(rd)

        z_recv = [
            rcopy(sb_ref.at[rows(0, c), :], g_ref.at[rows(q_row, c), :],
                  z_s.at[c], z_r.at[c], z_peer)
            for c in range(C)
        ]
        ze_recv = [
            rcopy(sb_ref.at[rows(0, 0), :], g_ref.at[rows(t_row, 0), :],
                  z_s.at[C + i], z_r.at[C + i], z_peer)
            for i, t_row in enumerate((qx_row, qy_row, qxy_row))
        ]
        xq_recv = {
            c: rcopy(sb_ref.at[rows(0, c), :], g_ref.at[rows(qx_row, c), :],
                     xq_s.at[c - 1], xq_r.at[c - 1], x_nbr)
            for c in range(1, C)
        }
        yq_recv = {
            c: rcopy(sb_ref.at[rows(0, c), :], g_ref.at[rows(qy_row, c), :],
                     yq_s.at[c - 1], yq_r.at[c - 1], y_nbr)
            for c in range(1, C)
        }
        yr_recv = {
            c: rcopy(sb_ref.at[rows(0, c), colA],
                     g_ref.at[rows(qxy_row, c), colA],
                     yr_s.at[c - 1], yr_r.at[c - 1], y_nbr)
            for c in range(1, C)
        }
        xr_recv = {
            c: rcopy(sb_ref.at[rows(0, c), colB],
                     g_ref.at[rows(qxy_row, c), colB],
                     xr_s.at[c - 1], xr_r.at[c - 1], x_nbr)
            for c in range(1, C)
        }

        def fwd_own(c):
            for ssem, rsem, dev in ((xq_s, xq_r, x_nbr), (yq_s, yq_r, y_nbr)):
                rd = rcopy(
                    g_ref.at[rows(q_row, c), :], g_ref.at[rows(q_row, c), :],
                    ssem.at[c - 1], rsem.at[c - 1], dev,
                )
                rd.start()
                sends.append(rd)

        def relay_qx(c):
            rd = rcopy(
                g_ref.at[rows(qx_row, c), colA], g_ref.at[rows(qx_row, c), colA],
                yr_s.at[c - 1], yr_r.at[c - 1], y_nbr,
            )
            rd.start()
            sends.append(rd)

        def relay_qy(c):
            rd = rcopy(
                g_ref.at[rows(qy_row, c), colB], g_ref.at[rows(qy_row, c), colB],
                xr_s.at[c - 1], xr_r.at[c - 1], x_nbr,
            )
            rd.start()
            sends.append(rd)

        for c in range(C):
            z_recv[c].wait_recv()
            if c >= 1:
                fwd_own(c)
            add_chunk(q_row, c)
            if c >= 2:
                cc = c - 1
                xq_recv[cc].wait_recv()
                relay_qx(cc)
                add_chunk(qx_row, cc)
                yq_recv[cc].wait_recv()
                relay_qy(cc)
                add_chunk(qy_row, cc)
        xq_recv[C - 1].wait_recv()
        relay_qx(C - 1)
        add_chunk(qx_row, C - 1)
        yq_recv[C - 1].wait_recv()
        relay_qy(C - 1)
        add_chunk(qy_row, C - 1)

        for rd, t_row in zip(ze_recv, (qx_row, qy_row, qxy_row)):
            rd.wait_recv()
            add_chunk(t_row, 0)

        for c in range(1, C):
            yr_recv[c].wait_recv()
            xr_recv[c].wait_recv()
            add_chunk(qxy_row, c)

        for rd in sends:
            rd.wait_send()

    return pl.pallas_call(
        body,
        out_shape=jax.ShapeDtypeStruct((m, n), jnp.float32),
        in_specs=[pl.BlockSpec(memory_space=pltpu.VMEM)],
        out_specs=pl.BlockSpec(memory_space=pltpu.VMEM),
        scratch_shapes=[
            pltpu.VMEM((m, n), jnp.float32),
            pltpu.VMEM((m // Q, n), jnp.float32),
            pltpu.SemaphoreType.DMA((C + 3,)),
            pltpu.SemaphoreType.DMA((C + 3,)),
            pltpu.SemaphoreType.DMA((C - 1,)),
            pltpu.SemaphoreType.DMA((C - 1,)),
            pltpu.SemaphoreType.DMA((C - 1,)),
            pltpu.SemaphoreType.DMA((C - 1,)),
            pltpu.SemaphoreType.DMA((C - 1,)),
            pltpu.SemaphoreType.DMA((C - 1,)),
            pltpu.SemaphoreType.DMA((C - 1,)),
            pltpu.SemaphoreType.DMA((C - 1,)),
        ],
        compiler_params=pltpu.CompilerParams(collective_id=0),
    )(x)
